# baseline (device time: 208173 ns/iter reference)
import jax
import jax.numpy as jnp
from jax import lax
from jax.experimental import pallas as pl
from jax.experimental.pallas import tpu as pltpu

N_DEV = 16


def kernel(x, w_mat):
    m_per, k = x.shape
    _, n_per = w_mat.shape
    m_glob = N_DEV * m_per

    def body(x_ref, w_ref, out_ref, comm_ref, send_sems, recv_sems):
        my = lax.axis_index("i")
        left = lax.rem(my + (N_DEV - 1), N_DEV)
        right = lax.rem(my + 1, N_DEV)

        barrier_sem = pltpu.get_barrier_semaphore()
        for nbr in (left, right):
            pl.semaphore_signal(
                barrier_sem, inc=1,
                device_id=(nbr,), device_id_type=pl.DeviceIdType.MESH,
            )
        pl.semaphore_wait(barrier_sem, 2)

        comm_ref[0] = x_ref[...]
        out_ref[pl.ds(my * m_per, m_per), :] = jnp.maximum(
            jnp.dot(x_ref[...], w_ref[...], preferred_element_type=jnp.float32),
            0.0,
        )

        for h in range(N_DEV - 1):
            rdma = pltpu.make_async_remote_copy(
                src_ref=comm_ref.at[h],
                dst_ref=comm_ref.at[h + 1],
                send_sem=send_sems.at[h],
                recv_sem=recv_sems.at[h],
                device_id=(right,),
                device_id_type=pl.DeviceIdType.MESH,
            )
            rdma.start()
            rdma.wait()

            origin = lax.rem(my + (N_DEV - h - 1), N_DEV)
            out_ref[pl.ds(origin * m_per, m_per), :] = jnp.maximum(
                jnp.dot(
                    comm_ref[h + 1], w_ref[...],
                    preferred_element_type=jnp.float32,
                ),
                0.0,
            )

    return pl.pallas_call(
        body,
        out_shape=jax.ShapeDtypeStruct((m_glob, n_per), jnp.float32),
        in_specs=[
            pl.BlockSpec(memory_space=pltpu.VMEM),
            pl.BlockSpec(memory_space=pltpu.VMEM),
        ],
        out_specs=pl.BlockSpec(memory_space=pltpu.VMEM),
        scratch_shapes=[
            pltpu.VMEM((N_DEV, m_per, k), jnp.float32),
            pltpu.SemaphoreType.DMA((N_DEV - 1,)),
            pltpu.SemaphoreType.DMA((N_DEV - 1,)),
        ],
        compiler_params=pltpu.CompilerParams(collective_id=0),
    )(x, w_mat)


# device time: 124168 ns/iter; 1.6765x vs baseline; 1.6765x over previous
import jax
import jax.numpy as jnp
from jax import lax
from jax.experimental import pallas as pl
from jax.experimental.pallas import tpu as pltpu

N_DEV = 16
R_HOPS = 8
L_HOPS = 7


def kernel(x, w_mat):
    m_per, k = x.shape
    _, n_per = w_mat.shape
    m_glob = N_DEV * m_per

    def body(x_ref, w_ref, out_ref, comm_ref, ss_r, rs_r, ss_l, rs_l):
        my = lax.axis_index("i")
        left = lax.rem(my + (N_DEV - 1), N_DEV)
        right = lax.rem(my + 1, N_DEV)

        barrier_sem = pltpu.get_barrier_semaphore()
        for nbr in (left, right):
            pl.semaphore_signal(
                barrier_sem, inc=1,
                device_id=(nbr,), device_id_type=pl.DeviceIdType.MESH,
            )
        pl.semaphore_wait(barrier_sem, 2)

        comm_ref[0] = x_ref[...]

        def right_rdma(h):
            return pltpu.make_async_remote_copy(
                src_ref=comm_ref.at[h],
                dst_ref=comm_ref.at[h + 1],
                send_sem=ss_r.at[h],
                recv_sem=rs_r.at[h],
                device_id=(right,),
                device_id_type=pl.DeviceIdType.MESH,
            )

        def left_rdma(t):
            return pltpu.make_async_remote_copy(
                src_ref=comm_ref.at[0 if t == 0 else N_DEV - t],
                dst_ref=comm_ref.at[N_DEV - 1 - t],
                send_sem=ss_l.at[t],
                recv_sem=rs_l.at[t],
                device_id=(left,),
                device_id_type=pl.DeviceIdType.MESH,
            )

        def gemm_to_out(chunk, origin):
            out_ref[pl.ds(origin * m_per, m_per), :] = jnp.maximum(
                jnp.dot(chunk, w_ref[...], preferred_element_type=jnp.float32),
                0.0,
            )

        rd_r = [right_rdma(h) for h in range(R_HOPS)]
        rd_l = [left_rdma(t) for t in range(L_HOPS)]
        rd_r[0].start()
        rd_l[0].start()

        gemm_to_out(x_ref[...], my)

        for h in range(R_HOPS):
            rd_r[h].wait()
            if h < L_HOPS:
                rd_l[h].wait()
            if h + 1 < R_HOPS:
                rd_r[h + 1].start()
            if h + 1 < L_HOPS:
                rd_l[h + 1].start()
            gemm_to_out(comm_ref[h + 1], lax.rem(my + (N_DEV - h - 1), N_DEV))
            if h < L_HOPS:
                gemm_to_out(
                    comm_ref[N_DEV - 1 - h], lax.rem(my + h + 1, N_DEV)
                )

    return pl.pallas_call(
        body,
        out_shape=jax.ShapeDtypeStruct((m_glob, n_per), jnp.float32),
        in_specs=[
            pl.BlockSpec(memory_space=pltpu.VMEM),
            pl.BlockSpec(memory_space=pltpu.VMEM),
        ],
        out_specs=pl.BlockSpec(memory_space=pltpu.VMEM),
        scratch_shapes=[
            pltpu.VMEM((N_DEV, m_per, k), jnp.float32),
            pltpu.SemaphoreType.DMA((R_HOPS,)),
            pltpu.SemaphoreType.DMA((R_HOPS,)),
            pltpu.SemaphoreType.DMA((L_HOPS,)),
            pltpu.SemaphoreType.DMA((L_HOPS,)),
        ],
        compiler_params=pltpu.CompilerParams(collective_id=0),
    )(x, w_mat)


# device time: 97655 ns/iter; 2.1317x vs baseline; 1.2715x over previous
import jax
import jax.numpy as jnp
from jax import lax
from jax.experimental import pallas as pl
from jax.experimental.pallas import tpu as pltpu

N_DEV = 16
HOPS = 8
HALVES = 2


def kernel(x, w_mat):
    m_per, k = x.shape
    _, n_per = w_mat.shape
    m_glob = N_DEV * m_per
    m_half = m_per // 2

    def r_has(h, s):
        return h < 7 or s == 0

    def l_has(h, s):
        return h < 7 or s == 1

    def body(x_ref, w_ref, out_ref, comm_ref, ss_r, rs_r, ss_l, rs_l):
        my = lax.axis_index("i")
        left = lax.rem(my + (N_DEV - 1), N_DEV)
        right = lax.rem(my + 1, N_DEV)

        barrier_sem = pltpu.get_barrier_semaphore()
        for nbr in (left, right):
            pl.semaphore_signal(
                barrier_sem, inc=1,
                device_id=(nbr,), device_id_type=pl.DeviceIdType.MESH,
            )
        pl.semaphore_wait(barrier_sem, 2)

        comm_ref[0, 0] = x_ref[:m_half]
        comm_ref[0, 1] = x_ref[m_half:]

        def r_rdma(h, s):
            return pltpu.make_async_remote_copy(
                src_ref=comm_ref.at[h, s],
                dst_ref=comm_ref.at[h + 1, s],
                send_sem=ss_r.at[h, s],
                recv_sem=rs_r.at[h, s],
                device_id=(right,),
                device_id_type=pl.DeviceIdType.MESH,
            )

        def l_rdma(t, s):
            return pltpu.make_async_remote_copy(
                src_ref=comm_ref.at[0 if t == 0 else N_DEV - t, s],
                dst_ref=comm_ref.at[N_DEV - 1 - t, s],
                send_sem=ss_l.at[t, s],
                recv_sem=rs_l.at[t, s],
                device_id=(left,),
                device_id_type=pl.DeviceIdType.MESH,
            )

        rd_r = [[r_rdma(h, s) if r_has(h, s) else None for s in range(HALVES)]
                for h in range(HOPS)]
        rd_l = [[l_rdma(t, s) if l_has(t, s) else None for s in range(HALVES)]
                for t in range(HOPS)]

        def gemm_half(idx, origin, s):
            out_ref[pl.ds(origin * m_per + s * m_half, m_half), :] = (
                jnp.maximum(
                    jnp.dot(
                        comm_ref[idx, s], w_ref[...],
                        preferred_element_type=jnp.float32,
                    ),
                    0.0,
                )
            )

        for s in range(HALVES):
            rd_r[0][s].start()
            rd_l[0][s].start()

        out_ref[pl.ds(my * m_per, m_per), :] = jnp.maximum(
            jnp.dot(x_ref[...], w_ref[...], preferred_element_type=jnp.float32),
            0.0,
        )

        for h in range(HOPS):
            for s in range(HALVES):
                if r_has(h, s):
                    rd_r[h][s].wait()
                    if h + 1 < HOPS and r_has(h + 1, s):
                        rd_r[h + 1][s].start()
                if l_has(h, s):
                    rd_l[h][s].wait()
                    if h + 1 < HOPS and l_has(h + 1, s):
                        rd_l[h + 1][s].start()
            if h < 7:
                origin_r = lax.rem(my + (N_DEV - h - 1), N_DEV)
                origin_l = lax.rem(my + h + 1, N_DEV)
                for s in range(HALVES):
                    gemm_half(h + 1, origin_r, s)
                    gemm_half(N_DEV - 1 - h, origin_l, s)
            else:
                origin = lax.rem(my + (N_DEV - 8), N_DEV)
                for s in range(HALVES):
                    gemm_half(8, origin, s)

    return pl.pallas_call(
        body,
        out_shape=jax.ShapeDtypeStruct((m_glob, n_per), jnp.float32),
        in_specs=[
            pl.BlockSpec(memory_space=pltpu.VMEM),
            pl.BlockSpec(memory_space=pltpu.VMEM),
        ],
        out_specs=pl.BlockSpec(memory_space=pltpu.VMEM),
        scratch_shapes=[
            pltpu.VMEM((N_DEV, HALVES, m_half, k), jnp.float32),
            pltpu.SemaphoreType.DMA((HOPS, HALVES)),
            pltpu.SemaphoreType.DMA((HOPS, HALVES)),
            pltpu.SemaphoreType.DMA((HOPS, HALVES)),
            pltpu.SemaphoreType.DMA((HOPS, HALVES)),
        ],
        compiler_params=pltpu.CompilerParams(collective_id=0),
    )(x, w_mat)


# device time: 97034 ns/iter; 2.1454x vs baseline; 1.0064x over previous
import jax
import jax.numpy as jnp
from jax import lax
from jax.experimental import pallas as pl
from jax.experimental.pallas import tpu as pltpu

N_DEV = 16
HOPS = 8
NQ = 4


def kernel(x, w_mat):
    m_per, k = x.shape
    _, n_per = w_mat.shape
    m_glob = N_DEV * m_per
    m_q = m_per // NQ

    def r_has(h, q):
        return h < 7 or q < 2

    def l_has(h, q):
        return h < 7 or q >= 2

    def body(x_ref, w_ref, out_ref, comm_ref, ss_r, rs_r, ss_l, rs_l):
        my = lax.axis_index("i")
        left = lax.rem(my + (N_DEV - 1), N_DEV)
        right = lax.rem(my + 1, N_DEV)

        barrier_sem = pltpu.get_barrier_semaphore()
        for nbr in (left, right):
            pl.semaphore_signal(
                barrier_sem, inc=1,
                device_id=(nbr,), device_id_type=pl.DeviceIdType.MESH,
            )
        pl.semaphore_wait(barrier_sem, 2)

        comm_ref[0] = x_ref[...].reshape(NQ, m_q, k)

        def r_rdma(h, q):
            return pltpu.make_async_remote_copy(
                src_ref=comm_ref.at[h, q],
                dst_ref=comm_ref.at[h + 1, q],
                send_sem=ss_r.at[h, q],
                recv_sem=rs_r.at[h, q],
                device_id=(right,),
                device_id_type=pl.DeviceIdType.MESH,
            )

        def l_rdma(t, q):
            return pltpu.make_async_remote_copy(
                src_ref=comm_ref.at[0 if t == 0 else N_DEV - t, q],
                dst_ref=comm_ref.at[N_DEV - 1 - t, q],
                send_sem=ss_l.at[t, q],
                recv_sem=rs_l.at[t, q],
                device_id=(left,),
                device_id_type=pl.DeviceIdType.MESH,
            )

        rd_r = [[r_rdma(h, q) if r_has(h, q) else None for q in range(NQ)]
                for h in range(HOPS)]
        rd_l = [[l_rdma(t, q) if l_has(t, q) else None for q in range(NQ)]
                for t in range(HOPS)]

        def gemm_chunk(idx, origin):
            out_ref[pl.ds(origin * m_per, m_per), :] = jnp.maximum(
                jnp.dot(
                    comm_ref[idx].reshape(m_per, k), w_ref[...],
                    preferred_element_type=jnp.float32,
                ),
                0.0,
            )

        for q in range(NQ):
            rd_r[0][q].start()
            rd_l[0][q].start()

        out_ref[pl.ds(my * m_per, m_per), :] = jnp.maximum(
            jnp.dot(x_ref[...], w_ref[...], preferred_element_type=jnp.float32),
            0.0,
        )

        for h in range(HOPS):
            for q in range(NQ):
                if r_has(h, q):
                    rd_r[h][q].wait_recv()
                    if h + 1 < HOPS and r_has(h + 1, q):
                        rd_r[h + 1][q].start()
                if l_has(h, q):
                    rd_l[h][q].wait_recv()
                    if h + 1 < HOPS and l_has(h + 1, q):
                        rd_l[h + 1][q].start()
            if h < 7:
                gemm_chunk(h + 1, lax.rem(my + (N_DEV - h - 1), N_DEV))
                gemm_chunk(N_DEV - 1 - h, lax.rem(my + h + 1, N_DEV))
            else:
                gemm_chunk(8, lax.rem(my + (N_DEV - 8), N_DEV))

        for h in range(HOPS):
            for q in range(NQ):
                if r_has(h, q):
                    rd_r[h][q].wait_send()
                if l_has(h, q):
                    rd_l[h][q].wait_send()

    return pl.pallas_call(
        body,
        out_shape=jax.ShapeDtypeStruct((m_glob, n_per), jnp.float32),
        in_specs=[
            pl.BlockSpec(memory_space=pltpu.VMEM),
            pl.BlockSpec(memory_space=pltpu.VMEM),
        ],
        out_specs=pl.BlockSpec(memory_space=pltpu.VMEM),
        scratch_shapes=[
            pltpu.VMEM((N_DEV, NQ, m_q, k), jnp.float32),
            pltpu.SemaphoreType.DMA((HOPS, NQ)),
            pltpu.SemaphoreType.DMA((HOPS, NQ)),
            pltpu.SemaphoreType.DMA((HOPS, NQ)),
            pltpu.SemaphoreType.DMA((HOPS, NQ)),
        ],
        compiler_params=pltpu.CompilerParams(collective_id=0),
    )(x, w_mat)


# device time: 96977 ns/iter; 2.1466x vs baseline; 1.0006x over previous
import jax
import jax.numpy as jnp
from jax import lax
from jax.experimental import pallas as pl
from jax.experimental.pallas import tpu as pltpu

N_DEV = 16
HOPS = 8
NQ = 4


def kernel(x, w_mat):
    m_per, k = x.shape
    _, n_per = w_mat.shape
    m_glob = N_DEV * m_per
    m_q = m_per // NQ

    def r_has(h, q):
        return h < 7 or q < 2

    def l_has(h, q):
        return h < 7 or q >= 2

    def body(x_ref, w_ref, out_ref, comm_ref, ss_r, rs_r, ss_l, rs_l):
        my = lax.axis_index("i")
        left = lax.rem(my + (N_DEV - 1), N_DEV)
        right = lax.rem(my + 1, N_DEV)

        barrier_sem = pltpu.get_barrier_semaphore()
        for nbr in (left, right):
            pl.semaphore_signal(
                barrier_sem, inc=1,
                device_id=(nbr,), device_id_type=pl.DeviceIdType.MESH,
            )
        pl.semaphore_wait(barrier_sem, 2)

        def r_rdma(h, q):
            src = (x_ref.at[pl.ds(q * m_q, m_q)] if h == 0
                   else comm_ref.at[h, q])
            return pltpu.make_async_remote_copy(
                src_ref=src,
                dst_ref=comm_ref.at[h + 1, q],
                send_sem=ss_r.at[h, q],
                recv_sem=rs_r.at[h, q],
                device_id=(right,),
                device_id_type=pl.DeviceIdType.MESH,
            )

        def l_rdma(t, q):
            src = (x_ref.at[pl.ds(q * m_q, m_q)] if t == 0
                   else comm_ref.at[N_DEV - t, q])
            return pltpu.make_async_remote_copy(
                src_ref=src,
                dst_ref=comm_ref.at[N_DEV - 1 - t, q],
                send_sem=ss_l.at[t, q],
                recv_sem=rs_l.at[t, q],
                device_id=(left,),
                device_id_type=pl.DeviceIdType.MESH,
            )

        rd_r = [[r_rdma(h, q) if r_has(h, q) else None for q in range(NQ)]
                for h in range(HOPS)]
        rd_l = [[l_rdma(t, q) if l_has(t, q) else None for q in range(NQ)]
                for t in range(HOPS)]

        def gemm_chunk(idx, origin):
            out_ref[pl.ds(origin * m_per, m_per), :] = jnp.maximum(
                jnp.dot(
                    comm_ref[idx].reshape(m_per, k), w_ref[...],
                    preferred_element_type=jnp.float32,
                ),
                0.0,
            )

        for q in range(NQ):
            rd_r[0][q].start()
            rd_l[0][q].start()

        out_ref[pl.ds(my * m_per, m_per), :] = jnp.maximum(
            jnp.dot(x_ref[...], w_ref[...], preferred_element_type=jnp.float32),
            0.0,
        )

        for h in range(HOPS):
            for q in range(NQ):
                if r_has(h, q):
                    rd_r[h][q].wait_recv()
                    if h + 1 < HOPS and r_has(h + 1, q):
                        rd_r[h + 1][q].start()
                if l_has(h, q):
                    rd_l[h][q].wait_recv()
                    if h + 1 < HOPS and l_has(h + 1, q):
                        rd_l[h + 1][q].start()
            if h < 7:
                gemm_chunk(h + 1, lax.rem(my + (N_DEV - h - 1), N_DEV))
                gemm_chunk(N_DEV - 1 - h, lax.rem(my + h + 1, N_DEV))
            else:
                gemm_chunk(8, lax.rem(my + (N_DEV - 8), N_DEV))

        for h in range(HOPS):
            for q in range(NQ):
                if r_has(h, q):
                    rd_r[h][q].wait_send()
                if l_has(h, q):
                    rd_l[h][q].wait_send()

    return pl.pallas_call(
        body,
        out_shape=jax.ShapeDtypeStruct((m_glob, n_per), jnp.float32),
        in_specs=[
            pl.BlockSpec(memory_space=pltpu.VMEM),
            pl.BlockSpec(memory_space=pltpu.VMEM),
        ],
        out_specs=pl.BlockSpec(memory_space=pltpu.VMEM),
        scratch_shapes=[
            pltpu.VMEM((N_DEV, NQ, m_q, k), jnp.float32),
            pltpu.SemaphoreType.DMA((HOPS, NQ)),
            pltpu.SemaphoreType.DMA((HOPS, NQ)),
            pltpu.SemaphoreType.DMA((HOPS, NQ)),
            pltpu.SemaphoreType.DMA((HOPS, NQ)),
        ],
        compiler_params=pltpu.CompilerParams(collective_id=0),
    )(x, w_mat)


# device time: 94110 ns/iter; 2.2120x vs baseline; 1.0305x over previous
import jax
import jax.numpy as jnp
from jax import lax
from jax.experimental import pallas as pl
from jax.experimental.pallas import tpu as pltpu

N_DEV = 16
HOPS = 8
NQ = 4


def kernel(x, w_mat):
    m_per, k = x.shape
    _, n_per = w_mat.shape
    m_glob = N_DEV * m_per
    m_q = m_per // NQ

    def r_has(h, q):
        return h < 7 or q < 2

    def l_has(h, q):
        return h < 7 or q >= 2

    def body(x_ref, w_ref, out_ref, comm_ref, ss_r, rs_r, ss_l, rs_l):
        my = lax.axis_index("i")

        def ring_of_logical(p):
            j = lax.rem(p, 4)
            z = p // 4
            w = jnp.where(lax.rem(j, 2) == 0, z, 3 - z)
            return 4 * j + w

        def logical_of_ring(kk):
            c = kk // 4
            w = lax.rem(kk, 4)
            z = jnp.where(lax.rem(c, 2) == 0, w, 3 - w)
            return 4 * z + c

        rr = ring_of_logical(my)
        left = logical_of_ring(lax.rem(rr + (N_DEV - 1), N_DEV))
        right = logical_of_ring(lax.rem(rr + 1, N_DEV))

        barrier_sem = pltpu.get_barrier_semaphore()
        for nbr in (left, right):
            pl.semaphore_signal(
                barrier_sem, inc=1,
                device_id=(nbr,), device_id_type=pl.DeviceIdType.MESH,
            )
        pl.semaphore_wait(barrier_sem, 2)

        def r_rdma(h, q):
            src = (x_ref.at[pl.ds(q * m_q, m_q)] if h == 0
                   else comm_ref.at[h, q])
            return pltpu.make_async_remote_copy(
                src_ref=src,
                dst_ref=comm_ref.at[h + 1, q],
                send_sem=ss_r.at[h, q],
                recv_sem=rs_r.at[h, q],
                device_id=(right,),
                device_id_type=pl.DeviceIdType.MESH,
            )

        def l_rdma(t, q):
            src = (x_ref.at[pl.ds(q * m_q, m_q)] if t == 0
                   else comm_ref.at[N_DEV - t, q])
            return pltpu.make_async_remote_copy(
                src_ref=src,
                dst_ref=comm_ref.at[N_DEV - 1 - t, q],
                send_sem=ss_l.at[t, q],
                recv_sem=rs_l.at[t, q],
                device_id=(left,),
                device_id_type=pl.DeviceIdType.MESH,
            )

        rd_r = [[r_rdma(h, q) if r_has(h, q) else None for q in range(NQ)]
                for h in range(HOPS)]
        rd_l = [[l_rdma(t, q) if l_has(t, q) else None for q in range(NQ)]
                for t in range(HOPS)]

        def gemm_chunk(idx, origin):
            out_ref[pl.ds(origin * m_per, m_per), :] = jnp.maximum(
                jnp.dot(
                    comm_ref[idx].reshape(m_per, k), w_ref[...],
                    preferred_element_type=jnp.float32,
                ),
                0.0,
            )

        for q in range(NQ):
            rd_r[0][q].start()
            rd_l[0][q].start()

        out_ref[pl.ds(my * m_per, m_per), :] = jnp.maximum(
            jnp.dot(x_ref[...], w_ref[...], preferred_element_type=jnp.float32),
            0.0,
        )

        for h in range(HOPS):
            for q in range(NQ):
                if r_has(h, q):
                    rd_r[h][q].wait_recv()
                    if h + 1 < HOPS and r_has(h + 1, q):
                        rd_r[h + 1][q].start()
                if l_has(h, q):
                    rd_l[h][q].wait_recv()
                    if h + 1 < HOPS and l_has(h + 1, q):
                        rd_l[h + 1][q].start()
            if h < 7:
                gemm_chunk(
                    h + 1,
                    logical_of_ring(lax.rem(rr + (N_DEV - h - 1), N_DEV)),
                )
                gemm_chunk(
                    N_DEV - 1 - h,
                    logical_of_ring(lax.rem(rr + h + 1, N_DEV)),
                )
            else:
                gemm_chunk(8, logical_of_ring(lax.rem(rr + 8, N_DEV)))

        for h in range(HOPS):
            for q in range(NQ):
                if r_has(h, q):
                    rd_r[h][q].wait_send()
                if l_has(h, q):
                    rd_l[h][q].wait_send()

    return pl.pallas_call(
        body,
        out_shape=jax.ShapeDtypeStruct((m_glob, n_per), jnp.float32),
        in_specs=[
            pl.BlockSpec(memory_space=pltpu.VMEM),
            pl.BlockSpec(memory_space=pltpu.VMEM),
        ],
        out_specs=pl.BlockSpec(memory_space=pltpu.VMEM),
        scratch_shapes=[
            pltpu.VMEM((N_DEV, NQ, m_q, k), jnp.float32),
            pltpu.SemaphoreType.DMA((HOPS, NQ)),
            pltpu.SemaphoreType.DMA((HOPS, NQ)),
            pltpu.SemaphoreType.DMA((HOPS, NQ)),
            pltpu.SemaphoreType.DMA((HOPS, NQ)),
        ],
        compiler_params=pltpu.CompilerParams(collective_id=0),
    )(x, w_mat)


# device time: 78008 ns/iter; 2.6686x vs baseline; 1.2064x over previous
import jax
import jax.numpy as jnp
from jax import lax
from jax.experimental import pallas as pl
from jax.experimental.pallas import tpu as pltpu

N_DEV = 16
NZ = 4
NF = 4
NQ = 4


def kernel(x, w_mat):
    m_per, k = x.shape
    _, n_per = w_mat.shape
    m_glob = N_DEV * m_per
    m_q = m_per // NQ

    def body(x_ref, w_ref, out_ref, gather_ref, copy_sem,
             su, ru, sd, rd,
             sA, rA, sB, rB,
             sC, rC, sD, rD):
        my = lax.axis_index("i")
        jj = lax.rem(my, NF)
        zz = my // NF
        j_left = lax.rem(jj + (NF - 1), NF)
        j_right = lax.rem(jj + 1, NF)
        p_right = NF * zz + j_right
        p_left = NF * zz + j_left
        p_up = my + NF
        p_dn = my - NF

        has_up = zz < NZ - 1
        has_dn = zz > 0

        started = []

        def start(desc, cond=None):
            if cond is None:
                desc.start()
            else:
                pl.when(cond)(lambda: desc.start())
            started.append((desc, cond))

        barrier_sem = pltpu.get_barrier_semaphore()
        for nbr in (p_left, p_right):
            pl.semaphore_signal(barrier_sem, inc=1, device_id=(nbr,),
                                device_id_type=pl.DeviceIdType.MESH)
        pl.when(has_up)(lambda: pl.semaphore_signal(
            barrier_sem, inc=1, device_id=(p_up,),
            device_id_type=pl.DeviceIdType.MESH))
        pl.when(has_dn)(lambda: pl.semaphore_signal(
            barrier_sem, inc=1, device_id=(p_dn,),
            device_id_type=pl.DeviceIdType.MESH))
        pl.semaphore_wait(barrier_sem, 2)
        pl.when(has_up)(lambda: pl.semaphore_wait(barrier_sem, 1))
        pl.when(has_dn)(lambda: pl.semaphore_wait(barrier_sem, 1))

        copies = [
            pltpu.make_async_copy(
                x_ref.at[pl.ds(q * m_q, m_q)],
                gather_ref.at[jj, zz, q],
                copy_sem,
            )
            for q in range(NQ)
        ]
        for c in copies:
            c.start()
        for c in copies:
            c.wait()

        def z_send(z_src, q, up):
            return pltpu.make_async_remote_copy(
                src_ref=gather_ref.at[jj, z_src, q],
                dst_ref=gather_ref.at[jj, z_src, q],
                send_sem=(su if up else sd).at[z_src, q],
                recv_sem=(ru if up else rd).at[z_src, q],
                device_id=(p_up if up else p_dn,),
                device_id_type=pl.DeviceIdType.MESH,
            )

        def face_send(j_src, z_src, q, to_right, sems):
            s_sem, r_sem = sems
            return pltpu.make_async_remote_copy(
                src_ref=gather_ref.at[j_src, z_src, q],
                dst_ref=gather_ref.at[j_src, z_src, q],
                send_sem=s_sem.at[z_src, q],
                recv_sem=r_sem.at[z_src, q],
                device_id=(p_right if to_right else p_left,),
                device_id_type=pl.DeviceIdType.MESH,
            )

        def gemm_rows(piece_rows, origin):
            out_ref[pl.ds(origin * m_per, m_per), :] = jnp.maximum(
                jnp.dot(piece_rows, w_ref[...],
                        preferred_element_type=jnp.float32),
                0.0,
            )

        def gemm_chunk(j_src, z_src):
            gemm_rows(
                gather_ref[j_src, z_src].reshape(m_per, k),
                NF * z_src + j_src,
            )

        for q in range(NQ):
            start(z_send(zz, q, up=True), has_up)
            start(z_send(zz, q, up=False), has_dn)
            start(face_send(jj, zz, q, True, (sA, rA)))
            start(face_send(jj, zz, q, False, (sB, rB)))

        gemm_rows(x_ref[...], my)

        for d in range(1, NZ):
            z_lo = zz - d
            z_hi = zz + d
            lo_ok = z_lo >= 0
            hi_ok = z_hi <= NZ - 1
            for q in range(NQ):
                def lo_work(q=q):
                    pltpu.make_async_remote_copy(
                        src_ref=gather_ref.at[jj, z_lo, q],
                        dst_ref=gather_ref.at[jj, z_lo, q],
                        send_sem=su.at[z_lo, q],
                        recv_sem=ru.at[z_lo, q],
                        device_id=(p_dn,),
                        device_id_type=pl.DeviceIdType.MESH,
                    ).wait_recv()
                pl.when(lo_ok)(lo_work)
                start(z_send(z_lo, q, up=True),
                      jnp.logical_and(lo_ok, has_up))
                start(face_send(jj, z_lo, q, True, (sA, rA)), lo_ok)
                start(face_send(jj, z_lo, q, False, (sB, rB)), lo_ok)

                def hi_work(q=q):
                    pltpu.make_async_remote_copy(
                        src_ref=gather_ref.at[jj, z_hi, q],
                        dst_ref=gather_ref.at[jj, z_hi, q],
                        send_sem=sd.at[z_hi, q],
                        recv_sem=rd.at[z_hi, q],
                        device_id=(p_up,),
                        device_id_type=pl.DeviceIdType.MESH,
                    ).wait_recv()
                pl.when(hi_ok)(hi_work)
                start(z_send(z_hi, q, up=False),
                      jnp.logical_and(hi_ok, has_dn))
                start(face_send(jj, z_hi, q, True, (sA, rA)), hi_ok)
                start(face_send(jj, z_hi, q, False, (sB, rB)), hi_ok)
            pl.when(lo_ok)(lambda z=z_lo: gemm_chunk(jj, z))
            pl.when(hi_ok)(lambda z=z_hi: gemm_chunk(jj, z))

        z_order = [(zz, None)] + [
            (zz - d, zz + d) for d in range(1, NZ)
        ]

        def plane_pieces(process):
            for z_src, z_alt in z_order:
                if z_alt is None:
                    process(z_src, None)
                else:
                    process(z_src, z_src >= 0)
                    process(z_alt, z_alt <= NZ - 1)

        def neighbor_stacks(z_src, cond):
            for q in range(NQ):
                def workA(q=q):
                    face_send(j_left, z_src, q, True, (sA, rA)).wait_recv()
                if cond is None:
                    workA()
                else:
                    pl.when(cond)(workA)
                if q < NQ // 2:
                    start(face_send(j_left, z_src, q, True, (sC, rC)),
                          cond)

                def workB(q=q):
                    face_send(j_right, z_src, q, False, (sB, rB)).wait_recv()
                if cond is None:
                    workB()
                else:
                    pl.when(cond)(workB)
                if q >= NQ // 2:
                    start(face_send(j_right, z_src, q, False, (sD, rD)),
                          cond)

            def gemms():
                gemm_chunk(j_left, z_src)
                gemm_chunk(j_right, z_src)
            if cond is None:
                gemms()
            else:
                pl.when(cond)(gemms)

        plane_pieces(neighbor_stacks)

        j_opp = lax.rem(jj + 2, NF)

        def opposite_halves(z_src, cond):
            for q in range(NQ):
                def work(q=q):
                    if q < NQ // 2:
                        face_send(j_opp, z_src, q, True, (sC, rC)).wait_recv()
                    else:
                        face_send(j_opp, z_src, q, False, (sD, rD)).wait_recv()
                if cond is None:
                    work()
                else:
                    pl.when(cond)(work)

            def gemms():
                gemm_chunk(j_opp, z_src)
            if cond is None:
                gemms()
            else:
                pl.when(cond)(gemms)

        plane_pieces(opposite_halves)

        for desc, cond in started:
            if cond is None:
                desc.wait_send()
            else:
                pl.when(cond)(lambda d=desc: d.wait_send())

    dma = pltpu.SemaphoreType.DMA
    return pl.pallas_call(
        body,
        out_shape=jax.ShapeDtypeStruct((m_glob, n_per), jnp.float32),
        in_specs=[
            pl.BlockSpec(memory_space=pltpu.VMEM),
            pl.BlockSpec(memory_space=pltpu.VMEM),
        ],
        out_specs=pl.BlockSpec(memory_space=pltpu.VMEM),
        scratch_shapes=[
            pltpu.VMEM((NF, NZ, NQ, m_q, k), jnp.float32),
            dma,
            dma((NZ, NQ)), dma((NZ, NQ)), dma((NZ, NQ)), dma((NZ, NQ)),
            dma((NZ, NQ)), dma((NZ, NQ)), dma((NZ, NQ)), dma((NZ, NQ)),
            dma((NZ, NQ)), dma((NZ, NQ)), dma((NZ, NQ)), dma((NZ, NQ)),
        ],
        compiler_params=pltpu.CompilerParams(collective_id=0),
    )(x, w_mat)
